# Initial kernel scaffold; baseline (speedup 1.0000x reference)
import jax
import jax.numpy as jnp
from jax.experimental import pallas as pl


def kernel(input):
    raise NotImplementedError("write your pallas kernel here")


# speedup vs baseline: 37.8145x; 37.8145x over previous
import functools
import jax
import jax.numpy as jnp
from jax import lax
from jax.experimental import pallas as pl
from jax.experimental.pallas import tpu as pltpu
from jax.experimental.pallas import tpu_sc as plsc

_A = 6.26
_KP = 64
_KN = 64
_N = 128 * 32768
_NW = 32
_SHARD = _N // _NW
_CHUNK = 4096
_NB = 2048
_BSHIFT = 19
_BOFF = 1008
_CCAP = 128
_MCAP = 96

_mesh = plsc.VectorSubcoreMesh(core_axis_name="c", subcore_axis_name="s")


def _wid():
    return lax.axis_index("s") * 2 + lax.axis_index("c")


def _bucket(key):
    b = lax.shift_right_logical(key, _BSHIFT) - _BOFF
    return jnp.clip(b, 0, _NB - 1)


@functools.partial(
    pl.kernel,
    mesh=_mesh,
    compiler_params=pltpu.CompilerParams(needs_layout_passes=False),
    out_type=(
        jax.ShapeDtypeStruct((_NW, _NB), jnp.int32),
        jax.ShapeDtypeStruct((_NW, _NB), jnp.int32),
        jax.ShapeDtypeStruct((_NW, 32), jnp.int32),
    ),
    scratch_types=[
        pltpu.VMEM((_CHUNK,), jnp.float32),
        pltpu.VMEM((_NB * 16,), jnp.int32),
        pltpu.VMEM((_NB * 16,), jnp.int32),
        pltpu.VMEM((_NB,), jnp.int32),
        pltpu.VMEM((_NB,), jnp.int32),
        pltpu.VMEM((32,), jnp.int32),
    ],
)
def _k1(x_hbm, hp_hbm, hn_hbm, cnt_hbm, chunk, hp, hn, rowp, rown, crow):
    w = _wid()
    base = w * _SHARD
    zero16 = jnp.zeros((16,), jnp.int32)
    ones = jnp.ones((16,), jnp.int32)
    lane = lax.iota(jnp.int32, 16)

    def zrow(i, _):
        hp[pl.ds(i * 16, 16)] = zero16
        hn[pl.ds(i * 16, 16)] = zero16
        return 0

    lax.fori_loop(0, _NB, zrow, 0)

    def chunk_body(c, carry):
        cp, cn = carry
        pltpu.sync_copy(x_hbm.at[pl.ds(base + c * _CHUNK, _CHUNK)], chunk)

        def vec_body(i, carry2):
            cp2, cn2 = carry2
            v = chunk[pl.ds(i * 16, 16)]
            bits = plsc.bitcast(v, jnp.int32)
            pos = v > 0.0
            neg = v < 0.0
            bp = _bucket(bits) * 16 + lane
            bn = _bucket(bits & jnp.int32(0x7FFFFFFF)) * 16 + lane
            plsc.addupdate_scatter(hp, [bp], ones, mask=pos)
            plsc.addupdate_scatter(hn, [bn], ones, mask=neg)
            cp2 = cp2 + jnp.where(pos, 1, 0).astype(jnp.int32)
            cn2 = cn2 + jnp.where(neg, 1, 0).astype(jnp.int32)
            return cp2, cn2

        return lax.fori_loop(0, _CHUNK // 16, vec_body, (cp, cn))

    cp, cn = lax.fori_loop(0, _SHARD // _CHUNK, chunk_body, (zero16, zero16))
    crow[pl.ds(0, 16)] = cp
    crow[pl.ds(16, 16)] = cn

    def fold(i, _):
        rows = i * 256 + lane * 16
        ap = jnp.zeros((16,), jnp.int32)
        an = jnp.zeros((16,), jnp.int32)
        for l in range(16):
            ap = ap + plsc.load_gather(hp, [rows + l])
            an = an + plsc.load_gather(hn, [rows + l])
        rowp[pl.ds(i * 16, 16)] = ap
        rown[pl.ds(i * 16, 16)] = an
        return 0

    lax.fori_loop(0, _NB // 16, fold, 0)
    pltpu.sync_copy(rowp, hp_hbm.at[w])
    pltpu.sync_copy(rown, hn_hbm.at[w])
    pltpu.sync_copy(crow, cnt_hbm.at[w])


@functools.partial(
    pl.kernel,
    mesh=_mesh,
    compiler_params=pltpu.CompilerParams(needs_layout_passes=False),
    out_type=(
        jax.ShapeDtypeStruct((_NW, _CCAP), jnp.int32),
        jax.ShapeDtypeStruct((_NW, _CCAP), jnp.int32),
        jax.ShapeDtypeStruct((_NW, _CCAP), jnp.int32),
        jax.ShapeDtypeStruct((_NW, _CCAP), jnp.int32),
    ),
    scratch_types=[
        pltpu.VMEM((_CHUNK,), jnp.float32),
        pltpu.VMEM((32,), jnp.int32),
        pltpu.VMEM((_CCAP,), jnp.int32),
        pltpu.VMEM((_CCAP,), jnp.int32),
        pltpu.VMEM((_CCAP,), jnp.int32),
        pltpu.VMEM((_CCAP,), jnp.int32),
    ],
)
def _k3(x_hbm, prm_hbm, kp_hbm, ip_hbm, kn_hbm, in_hbm,
        chunk, prm, kpv, ipv, knv, inv):
    w = _wid()
    base = w * _SHARD
    zero16 = jnp.zeros((16,), jnp.int32)
    lane = lax.iota(jnp.int32, 16)
    for b in range(_CCAP // 16):
        kpv[pl.ds(b * 16, 16)] = zero16
        ipv[pl.ds(b * 16, 16)] = zero16
        knv[pl.ds(b * 16, 16)] = zero16
        inv[pl.ds(b * 16, 16)] = zero16
    pltpu.sync_copy(prm_hbm, prm)
    bcut_p = prm[pl.ds(0, 16)]
    bcut_n = prm[pl.ds(16, 16)]

    def chunk_body(c, carry):
        pltpu.sync_copy(x_hbm.at[pl.ds(base + c * _CHUNK, _CHUNK)], chunk)

        def vec_body(i, carry2):
            wpp2, wpn2 = carry2
            v = chunk[pl.ds(i * 16, 16)]
            bits = plsc.bitcast(v, jnp.int32)
            pos = v > 0.0
            neg = v < 0.0
            key_n = bits & jnp.int32(0x7FFFFFFF)
            gi = base + c * _CHUNK + i * 16 + lane
            mp = pos & (_bucket(bits) >= bcut_p)
            mn = neg & (_bucket(key_n) >= bcut_n)
            op = jnp.minimum(wpp2, _CCAP - 16)
            on = jnp.minimum(wpn2, _CCAP - 16)
            plsc.store_compressed(kpv.at[pl.ds(op, 16)], bits, mask=mp)
            plsc.store_compressed(ipv.at[pl.ds(op, 16)], gi, mask=mp)
            plsc.store_compressed(knv.at[pl.ds(on, 16)], key_n, mask=mn)
            plsc.store_compressed(inv.at[pl.ds(on, 16)], gi, mask=mn)
            wpp2 = jnp.minimum(
                wpp2 + jnp.max(plsc.all_reduce_population_count(mp)),
                _CCAP - 16)
            wpn2 = jnp.minimum(
                wpn2 + jnp.max(plsc.all_reduce_population_count(mn)),
                _CCAP - 16)
            return wpp2, wpn2

        return lax.fori_loop(0, _CHUNK // 16, vec_body, carry)

    lax.fori_loop(0, _SHARD // _CHUNK, chunk_body,
                  (jnp.int32(0), jnp.int32(0)))
    pltpu.sync_copy(kpv, kp_hbm.at[w])
    pltpu.sync_copy(ipv, ip_hbm.at[w])
    pltpu.sync_copy(knv, kn_hbm.at[w])
    pltpu.sync_copy(inv, in_hbm.at[w])


@functools.partial(
    pl.kernel,
    mesh=_mesh,
    compiler_params=pltpu.CompilerParams(needs_layout_passes=False),
    out_type=(
        jax.ShapeDtypeStruct((_NW, _MCAP), jnp.int32),
        jax.ShapeDtypeStruct((_NW, _MCAP), jnp.int32),
        jax.ShapeDtypeStruct((_NW, 32), jnp.int32),
    ),
    scratch_types=[
        pltpu.VMEM((_CHUNK,), jnp.float32),
        pltpu.VMEM((64,), jnp.float32),
        pltpu.VMEM((_MCAP,), jnp.int32),
        pltpu.VMEM((_MCAP,), jnp.int32),
        pltpu.VMEM((32,), jnp.int32),
    ],
)
def _k5(x_hbm, prm_hbm, mp_hbm, mn_hbm, cnt_hbm, chunk, prm, mpv, mnv, crow):
    w = _wid()
    base = w * _SHARD
    zero16 = jnp.zeros((16,), jnp.int32)
    lane = lax.iota(jnp.int32, 16)
    for b in range(_MCAP // 16):
        mpv[pl.ds(b * 16, 16)] = zero16
        mnv[pl.ds(b * 16, 16)] = zero16
    pltpu.sync_copy(prm_hbm, prm)
    epos = prm[pl.ds(0, 16)]
    gcut = prm[pl.ds(16, 16)]
    eneg = prm[pl.ds(32, 16)]
    hcut = prm[pl.ds(48, 16)]

    def chunk_body(c, carry):
        pltpu.sync_copy(x_hbm.at[pl.ds(base + c * _CHUNK, _CHUNK)], chunk)

        def vec_body(i, carry2):
            wpp2, wpn2, cp2, cn2 = carry2
            v = chunk[pl.ds(i * 16, 16)]
            pos = v > 0.0
            neg = v < 0.0
            gi = base + c * _CHUNK + i * 16 + lane
            mp = pos & ((v + epos) == gcut)
            mn = neg & ((v + eneg) == hcut)
            op = jnp.minimum(wpp2, _MCAP - 16)
            on = jnp.minimum(wpn2, _MCAP - 16)
            plsc.store_compressed(mpv.at[pl.ds(op, 16)], gi, mask=mp)
            plsc.store_compressed(mnv.at[pl.ds(on, 16)], gi, mask=mn)
            cp2 = cp2 + jnp.where(mp, 1, 0).astype(jnp.int32)
            cn2 = cn2 + jnp.where(mn, 1, 0).astype(jnp.int32)
            wpp2 = jnp.minimum(
                wpp2 + jnp.max(plsc.all_reduce_population_count(mp)),
                _MCAP - 16)
            wpn2 = jnp.minimum(
                wpn2 + jnp.max(plsc.all_reduce_population_count(mn)),
                _MCAP - 16)
            return wpp2, wpn2, cp2, cn2

        return lax.fori_loop(0, _CHUNK // 16, vec_body, carry)

    _, _, cp, cn = lax.fori_loop(
        0, _SHARD // _CHUNK, chunk_body,
        (jnp.int32(0), jnp.int32(0), zero16, zero16))
    crow[pl.ds(0, 16)] = cp
    crow[pl.ds(16, 16)] = cn
    pltpu.sync_copy(mpv, mp_hbm.at[w])
    pltpu.sync_copy(mnv, mn_hbm.at[w])
    pltpu.sync_copy(crow, cnt_hbm.at[w])


def _k6_body(x_ref, pf_ref, pi_ref, o_ref):
    r = pl.program_id(0)
    x = x_ref[...]
    epos = pf_ref[0, 0]
    gcut = pf_ref[0, 1]
    eneg = pf_ref[0, 2]
    hcut = pf_ref[0, 3]
    condp = pf_ref[0, 4] > 0.0
    condn = pf_ref[0, 5] > 0.0
    idxcut = pi_ref[0, 0]
    kthn = pi_ref[0, 1]
    rows = x.shape[0]
    ridx = lax.broadcasted_iota(jnp.int32, x.shape, 0) + r * rows
    cidx = lax.broadcasted_iota(jnp.int32, x.shape, 1)
    gi = ridx * 1024 + cidx
    g = x + epos
    winp = (x > 0.0) & ((g > gcut) | ((g == gcut) & (gi <= idxcut)))
    outp = jnp.where(winp, g, 0.0)
    outn = jnp.where(gi == kthn, x + eneg, 0.0)
    out = jnp.where(x > 0.0, jnp.where(condp, outp, x),
                    jnp.where(x < 0.0, jnp.where(condn, outn, x), x))
    o_ref[...] = out


def _k6(x2d, pf, pi):
    rows = 256
    return pl.pallas_call(
        _k6_body,
        out_shape=jax.ShapeDtypeStruct(x2d.shape, jnp.float32),
        grid=(x2d.shape[0] // rows,),
        in_specs=[
            pl.BlockSpec((rows, 1024), lambda i: (i, 0)),
            pl.BlockSpec((8, 128), lambda i: (0, 0)),
            pl.BlockSpec((8, 128), lambda i: (0, 0)),
        ],
        out_specs=pl.BlockSpec((rows, 1024), lambda i: (i, 0)),
    )(x2d, pf, pi)


def _merge_top64(keys, idx, k):
    kf = keys.reshape(-1)
    jf = idx.reshape(-1)
    kv, p = lax.top_k(kf, k)
    return kv, jf[p]


def kernel(input):
    xf = jnp.ravel(input)
    histp, histn, counts = _k1(xf)
    num_pos = jnp.sum(counts[:, :16])
    num_neg = jnp.sum(counts[:, 16:])
    condp = num_pos > _KP
    condn = num_neg > _KN

    def cutoff(hist):
        h = jnp.sum(hist, axis=0)
        rev = jnp.cumsum(h[::-1])
        reach = rev >= _KP
        p = jnp.argmax(reach)
        return jnp.where(jnp.any(reach), _NB - 1 - p, 0).astype(jnp.int32)

    bp = cutoff(histp)
    bn = cutoff(histn)
    prm1 = jnp.concatenate([
        jnp.full((16,), bp, jnp.int32), jnp.full((16,), bn, jnp.int32)])
    ckp, cip, ckn, cin = _k3(xf, prm1)

    kp64, ip64 = _merge_top64(ckp, cip, _KP)
    kn64, in64 = _merge_top64(ckn, cin, _KN)
    v64 = lax.bitcast_convert_type(kp64[_KP - 1], jnp.float32)
    w64 = -lax.bitcast_convert_type(kn64[_KN - 1], jnp.float32)

    winm_p = jnp.zeros((_N,), bool).at[ip64].set(True)
    winm_n = jnp.zeros((_N,), bool).at[in64].set(True)
    S1 = jnp.sum(jnp.where((xf > 0) & (~winm_p), xf, 0.0))
    S2 = jnp.sum(jnp.where((xf < 0) & (~winm_n), -xf, 0.0))
    Epos = S1 * _A
    Eneg = S2 * _A * -1.0

    gcut = v64 + Epos
    hcut = w64 + Eneg
    vals64 = lax.bitcast_convert_type(kp64, jnp.float32)
    n_pos = _KP - jnp.sum((vals64 + Epos) > gcut)
    nvals64 = -lax.bitcast_convert_type(kn64, jnp.float32)
    n_neg = _KN - jnp.sum((nvals64 + Eneg) < hcut)

    prm2 = jnp.concatenate([
        jnp.full((16,), Epos, jnp.float32), jnp.full((16,), gcut, jnp.float32),
        jnp.full((16,), Eneg, jnp.float32), jnp.full((16,), hcut, jnp.float32)])
    midp, midn, mcnt = _k5(xf, prm2)

    def nth_index(members, cnts, n):
        c = jnp.cumsum(cnts)
        t = jnp.argmax(c >= n)
        prev = jnp.where(t > 0, c[jnp.maximum(t - 1, 0)], 0)
        r = jnp.clip(n - prev - 1, 0, _MCAP - 1)
        return members[t, r]

    cntp_t = jnp.sum(mcnt[:, :16], axis=1)
    cntn_t = jnp.sum(mcnt[:, 16:], axis=1)
    idxcut = nth_index(midp, cntp_t, n_pos)
    kthn = nth_index(midn, cntn_t, n_neg)

    pf = jnp.zeros((8, 128), jnp.float32).at[0, :6].set(
        jnp.stack([Epos, gcut, Eneg, hcut,
                   jnp.where(condp, 1.0, 0.0), jnp.where(condn, 1.0, 0.0)]))
    pi = jnp.zeros((8, 128), jnp.int32).at[0, :2].set(
        jnp.stack([jnp.where(condp, idxcut, -1),
                   jnp.where(condn, kthn, -1)]).astype(jnp.int32))
    out2d = _k6(xf.reshape(4096, 1024), pf, pi)
    return out2d.reshape(-1)


# speedup vs baseline: 38.7136x; 1.0238x over previous
import functools
import jax
import jax.numpy as jnp
from jax import lax
from jax.experimental import pallas as pl
from jax.experimental.pallas import tpu as pltpu
from jax.experimental.pallas import tpu_sc as plsc

_A = 6.26
_KP = 64
_KN = 64
_N = 128 * 32768
_NW = 32
_SHARD = _N // _NW
_CHUNK = 16384
_NB = 2048
_BSHIFT = 19
_BOFF = 1008
_CCAP = 128
_MCAP = 96

_mesh = plsc.VectorSubcoreMesh(core_axis_name="c", subcore_axis_name="s")


def _wid():
    return lax.axis_index("s") * 2 + lax.axis_index("c")


def _bucket(key):
    b = lax.shift_right_logical(key, _BSHIFT) - _BOFF
    return jnp.clip(b, 0, _NB - 1)


@functools.partial(
    pl.kernel,
    mesh=_mesh,
    compiler_params=pltpu.CompilerParams(needs_layout_passes=False),
    out_type=(
        jax.ShapeDtypeStruct((_NW, _NB), jnp.int32),
        jax.ShapeDtypeStruct((_NW, _NB), jnp.int32),
        jax.ShapeDtypeStruct((_NW, 32), jnp.int32),
    ),
    scratch_types=[
        pltpu.VMEM((_CHUNK,), jnp.float32),
        pltpu.VMEM((_NB * 16,), jnp.int32),
        pltpu.VMEM((_NB * 16,), jnp.int32),
        pltpu.VMEM((_NB,), jnp.int32),
        pltpu.VMEM((_NB,), jnp.int32),
        pltpu.VMEM((32,), jnp.int32),
    ],
)
def _k1(x_hbm, hp_hbm, hn_hbm, cnt_hbm, chunk, hp, hn, rowp, rown, crow):
    w = _wid()
    base = w * _SHARD
    zero16 = jnp.zeros((16,), jnp.int32)
    ones = jnp.ones((16,), jnp.int32)
    lane = lax.iota(jnp.int32, 16)

    def zrow(i, _):
        hp[pl.ds(i * 16, 16)] = zero16
        hn[pl.ds(i * 16, 16)] = zero16
        return 0

    lax.fori_loop(0, _NB, zrow, 0)

    def chunk_body(c, carry):
        cp, cn = carry
        pltpu.sync_copy(x_hbm.at[pl.ds(base + c * _CHUNK, _CHUNK)], chunk)

        def vec_body(i, carry2):
            cp2, cn2 = carry2
            v = chunk[pl.ds(i * 16, 16)]
            bits = plsc.bitcast(v, jnp.int32)
            pos = v > 0.0
            neg = v < 0.0
            bp = _bucket(bits) * 16 + lane
            bn = _bucket(bits & jnp.int32(0x7FFFFFFF)) * 16 + lane
            plsc.addupdate_scatter(hp, [bp], ones, mask=pos)
            plsc.addupdate_scatter(hn, [bn], ones, mask=neg)
            cp2 = cp2 + jnp.where(pos, 1, 0).astype(jnp.int32)
            cn2 = cn2 + jnp.where(neg, 1, 0).astype(jnp.int32)
            return cp2, cn2

        return lax.fori_loop(0, _CHUNK // 16, vec_body, (cp, cn), unroll=4)

    cp, cn = lax.fori_loop(0, _SHARD // _CHUNK, chunk_body, (zero16, zero16))
    crow[pl.ds(0, 16)] = cp
    crow[pl.ds(16, 16)] = cn

    def fold(i, _):
        rows = i * 256 + lane * 16
        ap = jnp.zeros((16,), jnp.int32)
        an = jnp.zeros((16,), jnp.int32)
        for l in range(16):
            ap = ap + plsc.load_gather(hp, [rows + l])
            an = an + plsc.load_gather(hn, [rows + l])
        rowp[pl.ds(i * 16, 16)] = ap
        rown[pl.ds(i * 16, 16)] = an
        return 0

    lax.fori_loop(0, _NB // 16, fold, 0)
    pltpu.sync_copy(rowp, hp_hbm.at[w])
    pltpu.sync_copy(rown, hn_hbm.at[w])
    pltpu.sync_copy(crow, cnt_hbm.at[w])


@functools.partial(
    pl.kernel,
    mesh=_mesh,
    compiler_params=pltpu.CompilerParams(needs_layout_passes=False),
    out_type=(
        jax.ShapeDtypeStruct((_NW, _CCAP), jnp.int32),
        jax.ShapeDtypeStruct((_NW, _CCAP), jnp.int32),
        jax.ShapeDtypeStruct((_NW, _CCAP), jnp.int32),
        jax.ShapeDtypeStruct((_NW, _CCAP), jnp.int32),
    ),
    scratch_types=[
        pltpu.VMEM((_CHUNK,), jnp.float32),
        pltpu.VMEM((32,), jnp.int32),
        pltpu.VMEM((_CCAP,), jnp.int32),
        pltpu.VMEM((_CCAP,), jnp.int32),
        pltpu.VMEM((_CCAP,), jnp.int32),
        pltpu.VMEM((_CCAP,), jnp.int32),
    ],
)
def _k3(x_hbm, prm_hbm, kp_hbm, ip_hbm, kn_hbm, in_hbm,
        chunk, prm, kpv, ipv, knv, inv):
    w = _wid()
    base = w * _SHARD
    zero16 = jnp.zeros((16,), jnp.int32)
    lane = lax.iota(jnp.int32, 16)
    for b in range(_CCAP // 16):
        kpv[pl.ds(b * 16, 16)] = zero16
        ipv[pl.ds(b * 16, 16)] = zero16
        knv[pl.ds(b * 16, 16)] = zero16
        inv[pl.ds(b * 16, 16)] = zero16
    pltpu.sync_copy(prm_hbm, prm)
    bcut_p = prm[pl.ds(0, 16)]
    bcut_n = prm[pl.ds(16, 16)]

    def chunk_body(c, carry):
        pltpu.sync_copy(x_hbm.at[pl.ds(base + c * _CHUNK, _CHUNK)], chunk)

        def vec_body(i, carry2):
            wpp2, wpn2 = carry2
            v = chunk[pl.ds(i * 16, 16)]
            bits = plsc.bitcast(v, jnp.int32)
            pos = v > 0.0
            neg = v < 0.0
            key_n = bits & jnp.int32(0x7FFFFFFF)
            gi = base + c * _CHUNK + i * 16 + lane
            mp = pos & (_bucket(bits) >= bcut_p)
            mn = neg & (_bucket(key_n) >= bcut_n)
            op = jnp.minimum(wpp2, _CCAP - 16)
            on = jnp.minimum(wpn2, _CCAP - 16)
            plsc.store_compressed(kpv.at[pl.ds(op, 16)], bits, mask=mp)
            plsc.store_compressed(ipv.at[pl.ds(op, 16)], gi, mask=mp)
            plsc.store_compressed(knv.at[pl.ds(on, 16)], key_n, mask=mn)
            plsc.store_compressed(inv.at[pl.ds(on, 16)], gi, mask=mn)
            wpp2 = jnp.minimum(
                wpp2 + jnp.max(plsc.all_reduce_population_count(mp)),
                _CCAP - 16)
            wpn2 = jnp.minimum(
                wpn2 + jnp.max(plsc.all_reduce_population_count(mn)),
                _CCAP - 16)
            return wpp2, wpn2

        return lax.fori_loop(0, _CHUNK // 16, vec_body, carry, unroll=4)

    lax.fori_loop(0, _SHARD // _CHUNK, chunk_body,
                  (jnp.int32(0), jnp.int32(0)))
    pltpu.sync_copy(kpv, kp_hbm.at[w])
    pltpu.sync_copy(ipv, ip_hbm.at[w])
    pltpu.sync_copy(knv, kn_hbm.at[w])
    pltpu.sync_copy(inv, in_hbm.at[w])


@functools.partial(
    pl.kernel,
    mesh=_mesh,
    compiler_params=pltpu.CompilerParams(needs_layout_passes=False),
    out_type=(
        jax.ShapeDtypeStruct((_NW, _MCAP), jnp.int32),
        jax.ShapeDtypeStruct((_NW, _MCAP), jnp.int32),
        jax.ShapeDtypeStruct((_NW, 32), jnp.int32),
    ),
    scratch_types=[
        pltpu.VMEM((_CHUNK,), jnp.float32),
        pltpu.VMEM((64,), jnp.float32),
        pltpu.VMEM((_MCAP,), jnp.int32),
        pltpu.VMEM((_MCAP,), jnp.int32),
        pltpu.VMEM((32,), jnp.int32),
    ],
)
def _k5(x_hbm, prm_hbm, mp_hbm, mn_hbm, cnt_hbm, chunk, prm, mpv, mnv, crow):
    w = _wid()
    base = w * _SHARD
    zero16 = jnp.zeros((16,), jnp.int32)
    lane = lax.iota(jnp.int32, 16)
    for b in range(_MCAP // 16):
        mpv[pl.ds(b * 16, 16)] = zero16
        mnv[pl.ds(b * 16, 16)] = zero16
    pltpu.sync_copy(prm_hbm, prm)
    epos = prm[pl.ds(0, 16)]
    gcut = prm[pl.ds(16, 16)]
    eneg = prm[pl.ds(32, 16)]
    hcut = prm[pl.ds(48, 16)]

    def chunk_body(c, carry):
        pltpu.sync_copy(x_hbm.at[pl.ds(base + c * _CHUNK, _CHUNK)], chunk)

        def vec_body(i, carry2):
            wpp2, wpn2, cp2, cn2 = carry2
            v = chunk[pl.ds(i * 16, 16)]
            pos = v > 0.0
            neg = v < 0.0
            gi = base + c * _CHUNK + i * 16 + lane
            mp = pos & ((v + epos) == gcut)
            mn = neg & ((v + eneg) == hcut)
            op = jnp.minimum(wpp2, _MCAP - 16)
            on = jnp.minimum(wpn2, _MCAP - 16)
            plsc.store_compressed(mpv.at[pl.ds(op, 16)], gi, mask=mp)
            plsc.store_compressed(mnv.at[pl.ds(on, 16)], gi, mask=mn)
            cp2 = cp2 + jnp.where(mp, 1, 0).astype(jnp.int32)
            cn2 = cn2 + jnp.where(mn, 1, 0).astype(jnp.int32)
            wpp2 = jnp.minimum(
                wpp2 + jnp.max(plsc.all_reduce_population_count(mp)),
                _MCAP - 16)
            wpn2 = jnp.minimum(
                wpn2 + jnp.max(plsc.all_reduce_population_count(mn)),
                _MCAP - 16)
            return wpp2, wpn2, cp2, cn2

        return lax.fori_loop(0, _CHUNK // 16, vec_body, carry, unroll=4)

    _, _, cp, cn = lax.fori_loop(
        0, _SHARD // _CHUNK, chunk_body,
        (jnp.int32(0), jnp.int32(0), zero16, zero16))
    crow[pl.ds(0, 16)] = cp
    crow[pl.ds(16, 16)] = cn
    pltpu.sync_copy(mpv, mp_hbm.at[w])
    pltpu.sync_copy(mnv, mn_hbm.at[w])
    pltpu.sync_copy(crow, cnt_hbm.at[w])


def _k6_body(x_ref, pf_ref, pi_ref, o_ref):
    r = pl.program_id(0)
    x = x_ref[...]
    epos = pf_ref[0, 0]
    gcut = pf_ref[0, 1]
    eneg = pf_ref[0, 2]
    hcut = pf_ref[0, 3]
    condp = pf_ref[0, 4] > 0.0
    condn = pf_ref[0, 5] > 0.0
    idxcut = pi_ref[0, 0]
    kthn = pi_ref[0, 1]
    rows = x.shape[0]
    ridx = lax.broadcasted_iota(jnp.int32, x.shape, 0) + r * rows
    cidx = lax.broadcasted_iota(jnp.int32, x.shape, 1)
    gi = ridx * 1024 + cidx
    g = x + epos
    winp = (x > 0.0) & ((g > gcut) | ((g == gcut) & (gi <= idxcut)))
    outp = jnp.where(winp, g, 0.0)
    outn = jnp.where(gi == kthn, x + eneg, 0.0)
    out = jnp.where(x > 0.0, jnp.where(condp, outp, x),
                    jnp.where(x < 0.0, jnp.where(condn, outn, x), x))
    o_ref[...] = out


def _k6(x2d, pf, pi):
    rows = 256
    return pl.pallas_call(
        _k6_body,
        out_shape=jax.ShapeDtypeStruct(x2d.shape, jnp.float32),
        grid=(x2d.shape[0] // rows,),
        in_specs=[
            pl.BlockSpec((rows, 1024), lambda i: (i, 0)),
            pl.BlockSpec((8, 128), lambda i: (0, 0)),
            pl.BlockSpec((8, 128), lambda i: (0, 0)),
        ],
        out_specs=pl.BlockSpec((rows, 1024), lambda i: (i, 0)),
    )(x2d, pf, pi)


def _merge_top64(keys, idx, k):
    kf = keys.reshape(-1)
    jf = idx.reshape(-1)
    kv, p = lax.top_k(kf, k)
    return kv, jf[p]


def kernel(input):
    xf = jnp.ravel(input)
    histp, histn, counts = _k1(xf)
    num_pos = jnp.sum(counts[:, :16])
    num_neg = jnp.sum(counts[:, 16:])
    condp = num_pos > _KP
    condn = num_neg > _KN

    def cutoff(hist):
        h = jnp.sum(hist, axis=0)
        rev = jnp.cumsum(h[::-1])
        reach = rev >= _KP
        p = jnp.argmax(reach)
        return jnp.where(jnp.any(reach), _NB - 1 - p, 0).astype(jnp.int32)

    bp = cutoff(histp)
    bn = cutoff(histn)
    prm1 = jnp.concatenate([
        jnp.full((16,), bp, jnp.int32), jnp.full((16,), bn, jnp.int32)])
    ckp, cip, ckn, cin = _k3(xf, prm1)

    kp64, ip64 = _merge_top64(ckp, cip, _KP)
    kn64, in64 = _merge_top64(ckn, cin, _KN)
    v64 = lax.bitcast_convert_type(kp64[_KP - 1], jnp.float32)
    w64 = -lax.bitcast_convert_type(kn64[_KN - 1], jnp.float32)

    winm_p = jnp.zeros((_N,), bool).at[ip64].set(True)
    winm_n = jnp.zeros((_N,), bool).at[in64].set(True)
    S1 = jnp.sum(jnp.where((xf > 0) & (~winm_p), xf, 0.0))
    S2 = jnp.sum(jnp.where((xf < 0) & (~winm_n), -xf, 0.0))
    Epos = S1 * _A
    Eneg = S2 * _A * -1.0

    gcut = v64 + Epos
    hcut = w64 + Eneg
    vals64 = lax.bitcast_convert_type(kp64, jnp.float32)
    n_pos = _KP - jnp.sum((vals64 + Epos) > gcut)
    nvals64 = -lax.bitcast_convert_type(kn64, jnp.float32)
    n_neg = _KN - jnp.sum((nvals64 + Eneg) < hcut)

    prm2 = jnp.concatenate([
        jnp.full((16,), Epos, jnp.float32), jnp.full((16,), gcut, jnp.float32),
        jnp.full((16,), Eneg, jnp.float32), jnp.full((16,), hcut, jnp.float32)])
    midp, midn, mcnt = _k5(xf, prm2)

    def nth_index(members, cnts, n):
        c = jnp.cumsum(cnts)
        t = jnp.argmax(c >= n)
        prev = jnp.where(t > 0, c[jnp.maximum(t - 1, 0)], 0)
        r = jnp.clip(n - prev - 1, 0, _MCAP - 1)
        return members[t, r]

    cntp_t = jnp.sum(mcnt[:, :16], axis=1)
    cntn_t = jnp.sum(mcnt[:, 16:], axis=1)
    idxcut = nth_index(midp, cntp_t, n_pos)
    kthn = nth_index(midn, cntn_t, n_neg)

    pf = jnp.zeros((8, 128), jnp.float32).at[0, :6].set(
        jnp.stack([Epos, gcut, Eneg, hcut,
                   jnp.where(condp, 1.0, 0.0), jnp.where(condn, 1.0, 0.0)]))
    pi = jnp.zeros((8, 128), jnp.int32).at[0, :2].set(
        jnp.stack([jnp.where(condp, idxcut, -1),
                   jnp.where(condn, kthn, -1)]).astype(jnp.int32))
    out2d = _k6(xf.reshape(4096, 1024), pf, pi)
    return out2d.reshape(-1)
